# baseline (speedup 1.0000x reference)
import functools

import jax
import jax.numpy as jnp
from jax import lax
from jax.experimental import pallas as pl
from jax.experimental.pallas import tpu as pltpu
from jax.experimental.pallas import tpu_sc as plsc

N = 10000
E = 320000
F = 128
NC, NS, L = 2, 16, 16
NW = NC * NS
K = 64
CPW = 160
E_PAD = NW * CPW * K
NPAD = 10240
RPT = NPAD // NS
NB = 4
NH = 4
HCH = CPW // NH


def _make_agg(with_deg: bool):
    out_types = [jax.ShapeDtypeStruct((NC, NPAD, F), jnp.float32)]
    if with_deg:
        out_types.append(jax.ShapeDtypeStruct((NC * NPAD,), jnp.float32))
    scratch = [
        pltpu.VMEM((HCH, K), jnp.int32),
        pltpu.VMEM((HCH, K), jnp.int32),
        pltpu.VMEM((NB, K, F), jnp.float32),
        pltpu.VMEM((RPT,), jnp.float32),
        pltpu.VMEM_SHARED((NPAD, F), jnp.float32),
        pltpu.VMEM_SHARED((NPAD,), jnp.float32),
    ] + [pltpu.SemaphoreType.DMA] * (2 * NB)
    mesh = plsc.VectorSubcoreMesh(
        core_axis_name="c", subcore_axis_name="s", num_cores=NC, num_subcores=NS)

    def body(feat, srcc, dstc, *rest):
        if with_deg:
            acc_out, deg_out, src_v, dst_v, rows, dbuf, acc, deg_sp, *sems = rest
        else:
            acc_out, src_v, dst_v, rows, dbuf, acc, deg_sp, *sems = rest
        gsems, ssems = sems[:NB], sems[NB:]
        c = lax.axis_index("c")
        s = lax.axis_index("s")
        w = c * NS + s

        z16 = jnp.zeros((L,), jnp.float32)

        @pl.loop(0, K)
        def _(i):
            for jj in range(F // L):
                rows[0, i, pl.ds(jj * L, L)] = z16

        for t in range(RPT // K):
            pltpu.sync_copy(rows.at[0], acc.at[pl.ds(s * RPT + t * K, K)])

        @pl.loop(0, RPT // L)
        def _(i):
            dbuf[pl.ds(i * L, L)] = z16

        pltpu.sync_copy(dbuf, deg_sp.at[pl.ds(s * RPT, RPT)])

        if with_deg:
            one16 = jnp.ones((L,), jnp.float32)
            for jj in range(K // L):
                dbuf[pl.ds(jj * L, L)] = one16

        plsc.subcore_barrier()

        for h in range(NH):
            pltpu.sync_copy(srcc.at[pl.ds(w * CPW + h * HCH, HCH)], src_v)
            pltpu.sync_copy(dstc.at[pl.ds(w * CPW + h * HCH, HCH)], dst_v)

            for b in range(NB):
                pltpu.async_copy(feat.at[src_v.at[b]], rows.at[b], gsems[b])

            @pl.loop(0, HCH // NB)
            def _(t):
                for b in range(NB):
                    j = t * NB + b

                    @pl.when(t > 0)
                    def _():
                        pltpu.make_async_copy(
                            rows.at[b], acc.at[dst_v.at[j]], ssems[b]).wait()
                        pltpu.async_copy(feat.at[src_v.at[j]], rows.at[b],
                                         gsems[b])
                for b in range(NB):
                    j = t * NB + b
                    pltpu.make_async_copy(
                        feat.at[src_v.at[j]], rows.at[b], gsems[b]).wait()
                    pltpu.async_copy(rows.at[b], acc.at[dst_v.at[j]],
                                     ssems[b], add=True)
                    if with_deg:
                        pltpu.sync_copy(dbuf.at[pl.ds(0, K)],
                                        deg_sp.at[dst_v.at[j]], add=True)

            for b in range(NB):
                pltpu.make_async_copy(
                    rows.at[b], acc.at[dst_v.at[HCH - NB + b]],
                    ssems[b]).wait()

        plsc.subcore_barrier()

        pltpu.sync_copy(acc.at[pl.ds(s * RPT, RPT)],
                        acc_out.at[c, pl.ds(s * RPT, RPT)])
        if with_deg:
            pltpu.sync_copy(deg_sp.at[pl.ds(s * RPT, RPT)],
                            deg_out.at[pl.ds(c * NPAD + s * RPT, RPT)])

    return pl.kernel(body,
                     out_type=tuple(out_types) if with_deg else out_types[0],
                     mesh=mesh, scratch_types=scratch)


_agg_deg = _make_agg(True)
_agg = _make_agg(False)


R = 1000
G = N // R


def _tc1_body(x_r, p_r, d_r, w1s_r, w1n_r, b1_r, w2s_r, w2n_r, b2_r,
              y2_r, z_r):
    deg = d_r[0] + d_r[1]
    inv = 1.0 / jnp.maximum(deg, 1.0)
    mean = (p_r[0] + p_r[1]) * inv
    h = (jnp.dot(x_r[...], w1s_r[...], preferred_element_type=jnp.float32)
         + jnp.dot(mean, w1n_r[...], preferred_element_type=jnp.float32)
         + b1_r[...])
    h = jnp.maximum(h, 0.0)
    y2_r[...] = jnp.dot(h, w2n_r[...], preferred_element_type=jnp.float32)
    z_r[...] = (jnp.dot(h, w2s_r[...], preferred_element_type=jnp.float32)
                + b2_r[...])


_tc1 = pl.pallas_call(
    _tc1_body,
    grid=(G,),
    in_specs=[
        pl.BlockSpec((R, F), lambda i: (i, 0)),
        pl.BlockSpec((NC, R, F), lambda i: (0, i, 0)),
        pl.BlockSpec((NC, R, 1), lambda i: (0, i, 0)),
        pl.BlockSpec((128, 256), lambda i: (0, 0)),
        pl.BlockSpec((128, 256), lambda i: (0, 0)),
        pl.BlockSpec((1, 256), lambda i: (0, 0)),
        pl.BlockSpec((256, 128), lambda i: (0, 0)),
        pl.BlockSpec((256, 128), lambda i: (0, 0)),
        pl.BlockSpec((1, 128), lambda i: (0, 0)),
    ],
    out_specs=[pl.BlockSpec((R, F), lambda i: (i, 0)),
               pl.BlockSpec((R, F), lambda i: (i, 0))],
    out_shape=[jax.ShapeDtypeStruct((N, F), jnp.float32),
               jax.ShapeDtypeStruct((N, F), jnp.float32)],
)


def _tc2_body(z_r, p_r, d_r, o_r):
    deg = d_r[0] + d_r[1]
    inv = 1.0 / jnp.maximum(deg, 1.0)
    o_r[...] = z_r[...] + (p_r[0] + p_r[1]) * inv


_tc2 = pl.pallas_call(
    _tc2_body,
    grid=(G,),
    in_specs=[
        pl.BlockSpec((R, F), lambda i: (i, 0)),
        pl.BlockSpec((NC, R, F), lambda i: (0, i, 0)),
        pl.BlockSpec((NC, R, 1), lambda i: (0, i, 0)),
    ],
    out_specs=pl.BlockSpec((R, F), lambda i: (i, 0)),
    out_shape=jax.ShapeDtypeStruct((N, F), jnp.float32),
)


def kernel(x, edge_index, W1_self, W1_neigh, b1, W2_self, W2_neigh, b2):
    ei = edge_index.astype(jnp.int32)
    src, dst = ei[0], ei[1]
    pad = E_PAD - E
    srcc = jnp.concatenate([src, jnp.zeros((pad,), jnp.int32)]).reshape(E_PAD // K, K)
    dstc = jnp.concatenate([dst, jnp.full((pad,), N, jnp.int32)]).reshape(E_PAD // K, K)

    acc1, degp = _agg_deg(x, srcc, dstc)
    degp3 = degp.reshape(NC, NPAD, 1)
    y2, z = _tc1(x, acc1, degp3, W1_self, W1_neigh, b1.reshape(1, -1),
                 W2_self, W2_neigh, b2.reshape(1, -1))
    acc2 = _agg(y2, srcc, dstc)
    return _tc2(z, acc2, degp3)


# speedup vs baseline: 1.0235x; 1.0235x over previous
import functools

import jax
import jax.numpy as jnp
from jax import lax
from jax.experimental import pallas as pl
from jax.experimental.pallas import tpu as pltpu
from jax.experimental.pallas import tpu_sc as plsc

N = 10000
E = 320000
F = 128
NC, NS, L = 2, 16, 16
NW = NC * NS
K = 128
CPW = 80
E_PAD = NW * CPW * K
NPAD = 10240
RPT = NPAD // NS
NB = 2
NH = 2
HCH = CPW // NH


def _make_agg(with_deg: bool):
    out_types = [jax.ShapeDtypeStruct((NC, NPAD, F), jnp.float32)]
    if with_deg:
        out_types.append(jax.ShapeDtypeStruct((NC * NPAD,), jnp.float32))
    scratch = [
        pltpu.VMEM((HCH, K), jnp.int32),
        pltpu.VMEM((HCH, K), jnp.int32),
        pltpu.VMEM((NB, K, F), jnp.float32),
        pltpu.VMEM((RPT,), jnp.float32),
        pltpu.VMEM_SHARED((NPAD, F), jnp.float32),
        pltpu.VMEM_SHARED((NPAD,), jnp.float32),
    ] + [pltpu.SemaphoreType.DMA] * (2 * NB)
    mesh = plsc.VectorSubcoreMesh(
        core_axis_name="c", subcore_axis_name="s", num_cores=NC, num_subcores=NS)

    def body(feat, srcc, dstc, *rest):
        if with_deg:
            acc_out, deg_out, src_v, dst_v, rows, dbuf, acc, deg_sp, *sems = rest
        else:
            acc_out, src_v, dst_v, rows, dbuf, acc, deg_sp, *sems = rest
        gsems, ssems = sems[:NB], sems[NB:]
        c = lax.axis_index("c")
        s = lax.axis_index("s")
        w = c * NS + s

        z16 = jnp.zeros((L,), jnp.float32)

        @pl.loop(0, K)
        def _(i):
            for jj in range(F // L):
                rows[0, i, pl.ds(jj * L, L)] = z16

        for t in range(RPT // K):
            pltpu.sync_copy(rows.at[0], acc.at[pl.ds(s * RPT + t * K, K)])

        @pl.loop(0, RPT // L)
        def _(i):
            dbuf[pl.ds(i * L, L)] = z16

        pltpu.sync_copy(dbuf, deg_sp.at[pl.ds(s * RPT, RPT)])

        if with_deg:
            one16 = jnp.ones((L,), jnp.float32)
            for jj in range(K // L):
                dbuf[pl.ds(jj * L, L)] = one16

        plsc.subcore_barrier()

        for h in range(NH):
            pltpu.sync_copy(srcc.at[pl.ds(w * CPW + h * HCH, HCH)], src_v)
            pltpu.sync_copy(dstc.at[pl.ds(w * CPW + h * HCH, HCH)], dst_v)

            for b in range(NB):
                pltpu.async_copy(feat.at[src_v.at[b]], rows.at[b], gsems[b])

            @pl.loop(0, HCH // NB)
            def _(t):
                for b in range(NB):
                    j = t * NB + b

                    @pl.when(t > 0)
                    def _():
                        pltpu.make_async_copy(
                            rows.at[b], acc.at[dst_v.at[j]], ssems[b]).wait()
                        pltpu.async_copy(feat.at[src_v.at[j]], rows.at[b],
                                         gsems[b])
                for b in range(NB):
                    j = t * NB + b
                    pltpu.make_async_copy(
                        feat.at[src_v.at[j]], rows.at[b], gsems[b]).wait()
                    pltpu.async_copy(rows.at[b], acc.at[dst_v.at[j]],
                                     ssems[b], add=True)
                    if with_deg:
                        pltpu.sync_copy(dbuf.at[pl.ds(0, K)],
                                        deg_sp.at[dst_v.at[j]], add=True)

            for b in range(NB):
                pltpu.make_async_copy(
                    rows.at[b], acc.at[dst_v.at[HCH - NB + b]],
                    ssems[b]).wait()

        plsc.subcore_barrier()

        pltpu.sync_copy(acc.at[pl.ds(s * RPT, RPT)],
                        acc_out.at[c, pl.ds(s * RPT, RPT)])
        if with_deg:
            pltpu.sync_copy(deg_sp.at[pl.ds(s * RPT, RPT)],
                            deg_out.at[pl.ds(c * NPAD + s * RPT, RPT)])

    return pl.kernel(body,
                     out_type=tuple(out_types) if with_deg else out_types[0],
                     mesh=mesh, scratch_types=scratch)


_agg_deg = _make_agg(True)
_agg = _make_agg(False)


R = 1000
G = N // R


def _tc1_body(x_r, p_r, d_r, w1s_r, w1n_r, b1_r, w2s_r, w2n_r, b2_r,
              y2_r, z_r):
    deg = d_r[0] + d_r[1]
    inv = 1.0 / jnp.maximum(deg, 1.0)
    mean = (p_r[0] + p_r[1]) * inv
    h = (jnp.dot(x_r[...], w1s_r[...], preferred_element_type=jnp.float32)
         + jnp.dot(mean, w1n_r[...], preferred_element_type=jnp.float32)
         + b1_r[...])
    h = jnp.maximum(h, 0.0)
    y2_r[...] = jnp.dot(h, w2n_r[...], preferred_element_type=jnp.float32)
    z_r[...] = (jnp.dot(h, w2s_r[...], preferred_element_type=jnp.float32)
                + b2_r[...])


_tc1 = pl.pallas_call(
    _tc1_body,
    grid=(G,),
    in_specs=[
        pl.BlockSpec((R, F), lambda i: (i, 0)),
        pl.BlockSpec((NC, R, F), lambda i: (0, i, 0)),
        pl.BlockSpec((NC, R, 1), lambda i: (0, i, 0)),
        pl.BlockSpec((128, 256), lambda i: (0, 0)),
        pl.BlockSpec((128, 256), lambda i: (0, 0)),
        pl.BlockSpec((1, 256), lambda i: (0, 0)),
        pl.BlockSpec((256, 128), lambda i: (0, 0)),
        pl.BlockSpec((256, 128), lambda i: (0, 0)),
        pl.BlockSpec((1, 128), lambda i: (0, 0)),
    ],
    out_specs=[pl.BlockSpec((R, F), lambda i: (i, 0)),
               pl.BlockSpec((R, F), lambda i: (i, 0))],
    out_shape=[jax.ShapeDtypeStruct((N, F), jnp.float32),
               jax.ShapeDtypeStruct((N, F), jnp.float32)],
)


def _tc2_body(z_r, p_r, d_r, o_r):
    deg = d_r[0] + d_r[1]
    inv = 1.0 / jnp.maximum(deg, 1.0)
    o_r[...] = z_r[...] + (p_r[0] + p_r[1]) * inv


_tc2 = pl.pallas_call(
    _tc2_body,
    grid=(G,),
    in_specs=[
        pl.BlockSpec((R, F), lambda i: (i, 0)),
        pl.BlockSpec((NC, R, F), lambda i: (0, i, 0)),
        pl.BlockSpec((NC, R, 1), lambda i: (0, i, 0)),
    ],
    out_specs=pl.BlockSpec((R, F), lambda i: (i, 0)),
    out_shape=jax.ShapeDtypeStruct((N, F), jnp.float32),
)


def kernel(x, edge_index, W1_self, W1_neigh, b1, W2_self, W2_neigh, b2):
    ei = edge_index.astype(jnp.int32)
    src, dst = ei[0], ei[1]
    pad = E_PAD - E
    srcc = jnp.concatenate([src, jnp.zeros((pad,), jnp.int32)]).reshape(E_PAD // K, K)
    dstc = jnp.concatenate([dst, jnp.full((pad,), N, jnp.int32)]).reshape(E_PAD // K, K)

    acc1, degp = _agg_deg(x, srcc, dstc)
    degp3 = degp.reshape(NC, NPAD, 1)
    y2, z = _tc1(x, acc1, degp3, W1_self, W1_neigh, b1.reshape(1, -1),
                 W2_self, W2_neigh, b2.reshape(1, -1))
    acc2 = _agg(y2, srcc, dstc)
    return _tc2(z, acc2, degp3)
